# baseline (speedup 1.0000x reference)
import jax
import jax.numpy as jnp
from jax import lax
from jax.experimental import pallas as pl
from jax.experimental.pallas import tpu as pltpu
from jax.experimental.pallas import tpu_sc as plsc

N = 16777216
M = 1048576
NW = 32
RANGE = N // NW
W = 32768
NWIN = RANGE // W
TOTWIN = NW * NWIN
CAP = 2048
SLEN = CAP + 16
SRCMAX = M - SLEN
PADV = 0x7F000000


def _gather16(vec, idxs):
    dnums = lax.GatherDimensionNumbers(
        offset_dims=(), collapsed_slice_dims=(0,), start_index_map=(0,))
    return lax.gather(vec, idxs[:, None], dimension_numbers=dnums,
                      slice_sizes=(1,),
                      mode=lax.GatherScatterMode.PROMISE_IN_BOUNDS)


def _scatter_body(x_hbm, sidx_hbm, sval_hbm, starts_hbm, ends_hbm, acc_hbm,
                  out_hbm, winbuf, idxbuf, valbuf, sbuf, ebuf, accbuf):
    wid = lax.axis_index("s") * 2 + lax.axis_index("c")
    lanes = lax.iota(jnp.int32, 16)

    tb = pl.multiple_of(wid * 32, 8)
    pltpu.sync_copy(starts_hbm.at[pl.ds(tb, 32)], sbuf)
    pltpu.sync_copy(ends_hbm.at[pl.ds(tb, 32)], ebuf)
    pltpu.sync_copy(acc_hbm, accbuf)
    idxbuf[pl.ds(SLEN, 16)] = jnp.full((16,), PADV, jnp.int32)

    def make_window_body(accumulate_flag):
        def window_body(w, carry_w):
            gbase = pl.multiple_of(wid * RANGE + w * W, 8)
            pltpu.sync_copy(x_hbm.at[pl.ds(gbase, W)], winbuf)

            p_lo = sbuf[pl.ds(w, 16)][0]
            p_hi = ebuf[pl.ds(w, 16)][0]
            a_start = pl.multiple_of(p_lo & ~7, 8)
            nb = (p_hi - a_start + CAP - 1) // CAP

            def batch_body(b, carry):
                src0 = pl.multiple_of(a_start + b * CAP, 8)
                src = pl.multiple_of(jnp.minimum(src0, SRCMAX), 8)
                delta = src0 - src
                pltpu.sync_copy(sidx_hbm.at[pl.ds(src, SLEN)],
                                idxbuf.at[pl.ds(0, SLEN)])
                pltpu.sync_copy(sval_hbm.at[pl.ds(src, SLEN)],
                                valbuf.at[pl.ds(0, SLEN)])
                todo = jnp.minimum(p_hi - src0, CAP)
                nch = (todo + 15) // 16

                def chunk_body(c, carry2):
                    i0 = delta + c * 16
                    a = idxbuf[pl.ds(i0, 16)]
                    an = idxbuf[pl.ds(i0 + 1, 16)]
                    v = valbuf[pl.ds(i0, 16)]
                    local = a - gbase
                    inwin = (local >= 0) & (local < W)
                    runend = a != an
                    lclamp = jnp.minimum(jnp.maximum(local, 0), W - 1)
                    if not accumulate_flag:
                        plsc.store_scatter(winbuf, [lclamp], v,
                                           mask=runend & inwin)
                        return carry2
                    sv_ = v
                    for d in (1, 2, 4, 8):
                        srcl = jnp.maximum(lanes - d, 0)
                        vsh = _gather16(sv_, srcl)
                        ash = _gather16(a, srcl)
                        cond = (lanes >= d) & (a == ash)
                        sv_ = jnp.where(cond, sv_ + vsh, sv_)
                    carry_val, carry_idx = carry2
                    first_idx = jnp.sum(jnp.where(lanes == 0, a, 0), axis=0)
                    cont = carry_idx == first_idx
                    headmask = (a == first_idx) & cont
                    sv_ = jnp.where(headmask, sv_ + carry_val, sv_)
                    plsc.addupdate_scatter(winbuf, [lclamp], sv_,
                                           mask=runend & inwin)
                    last_val = jnp.sum(jnp.where(lanes == 15, sv_,
                                                 jnp.float32(0.0)), axis=0)
                    last_idx = jnp.sum(jnp.where(lanes == 15, a, 0), axis=0)
                    last_end = jnp.sum(jnp.where(lanes == 15,
                                                 runend.astype(jnp.int32),
                                                 0), axis=0)
                    new_cv = jnp.where(last_end == 1, jnp.float32(0.0),
                                       last_val)
                    new_ci = jnp.where(last_end == 1, jnp.int32(-1),
                                       last_idx)
                    return (new_cv, new_ci)

                return lax.fori_loop(0, nch, chunk_body, carry)

            carry0 = (jnp.float32(0.0), jnp.int32(-1))
            lax.fori_loop(0, nb, batch_body, carry0)
            pltpu.sync_copy(winbuf, out_hbm.at[pl.ds(gbase, W)])
            return carry_w
        return window_body

    acc = accbuf[pl.ds(0, 16)][0]
    lax.cond(acc != 0,
             lambda: lax.fori_loop(0, NWIN, make_window_body(True), 0),
             lambda: lax.fori_loop(0, NWIN, make_window_body(False), 0))


def _make_sc_kernel():
    mesh = plsc.VectorSubcoreMesh(core_axis_name="c", subcore_axis_name="s")
    return pl.kernel(
        _scatter_body,
        out_type=jax.ShapeDtypeStruct((N,), jnp.float32),
        mesh=mesh,
        compiler_params=pltpu.CompilerParams(needs_layout_passes=False),
        scratch_types=[
            pltpu.VMEM((W,), jnp.float32),
            pltpu.VMEM((SLEN + 16,), jnp.int32),
            pltpu.VMEM((SLEN + 16,), jnp.float32),
            pltpu.VMEM((32,), jnp.int32),
            pltpu.VMEM((32,), jnp.int32),
            pltpu.VMEM((16,), jnp.int32),
        ],
    )


def kernel(x, index, values, accumulate):
    idx32 = index.astype(jnp.int32)
    s_idx, s_val = lax.sort((idx32, values), num_keys=1, is_stable=False)

    win_starts = (jnp.arange(TOTWIN, dtype=jnp.int32) * W)
    b = jnp.searchsorted(s_idx, win_starts, side="left").astype(jnp.int32)
    e = jnp.concatenate([b[1:], jnp.array([M], jnp.int32)])
    starts = jnp.pad(b.reshape(NW, NWIN), ((0, 0), (0, 16))).reshape(-1)
    ends = jnp.pad(e.reshape(NW, NWIN), ((0, 0), (0, 16))).reshape(-1)
    accv = jnp.full((16,), jnp.asarray(accumulate).astype(jnp.int32))

    sc_k = _make_sc_kernel()
    return sc_k(x, s_idx, s_val, starts, ends, accv)


# speedup vs baseline: 1.0279x; 1.0279x over previous
import jax
import jax.numpy as jnp
from jax import lax
from jax.experimental import pallas as pl
from jax.experimental.pallas import tpu as pltpu
from jax.experimental.pallas import tpu_sc as plsc

N = 16777216
M = 1048576
NW = 32
RANGE = N // NW
W = 32768
NWIN = RANGE // W
TOTWIN = NW * NWIN
CAP = 2048
SLEN = CAP + 16
SRCMAX = M - SLEN
PADV = 0x7F000000


def _gather16(vec, idxs):
    dnums = lax.GatherDimensionNumbers(
        offset_dims=(), collapsed_slice_dims=(0,), start_index_map=(0,))
    return lax.gather(vec, idxs[:, None], dimension_numbers=dnums,
                      slice_sizes=(1,),
                      mode=lax.GatherScatterMode.PROMISE_IN_BOUNDS)


def _apply_pairs_set(winbuf, idxbuf, valbuf, sidx_hbm, sval_hbm,
                     gbase, p_lo, p_hi):
    a_start = pl.multiple_of(p_lo & ~7, 8)
    nb = (p_hi - a_start + CAP - 1) // CAP

    def batch_body(b, carry):
        src0 = pl.multiple_of(a_start + b * CAP, 8)
        src = pl.multiple_of(jnp.minimum(src0, SRCMAX), 8)
        delta = src0 - src
        pltpu.sync_copy(sidx_hbm.at[pl.ds(src, SLEN)],
                        idxbuf.at[pl.ds(0, SLEN)])
        pltpu.sync_copy(sval_hbm.at[pl.ds(src, SLEN)],
                        valbuf.at[pl.ds(0, SLEN)])
        todo = jnp.minimum(p_hi - src0, CAP)
        nch = (todo + 15) // 16

        def chunk_body(c, carry2):
            i0 = delta + c * 16
            a = idxbuf[pl.ds(i0, 16)]
            an = idxbuf[pl.ds(i0 + 1, 16)]
            v = valbuf[pl.ds(i0, 16)]
            local = a - gbase
            inwin = (local >= 0) & (local < W)
            runend = a != an
            lclamp = jnp.minimum(jnp.maximum(local, 0), W - 1)
            plsc.store_scatter(winbuf, [lclamp], v, mask=runend & inwin)
            return carry2

        return lax.fori_loop(0, nch, chunk_body, carry)

    lax.fori_loop(0, nb, batch_body, 0)


def _scatter_body(x_hbm, sidx_hbm, sval_hbm, starts_hbm, ends_hbm, acc_hbm,
                  out_hbm, winbuf0, winbuf1, idxbuf, valbuf, sbuf, ebuf,
                  accbuf, si0, si1, so0, so1):
    wid = lax.axis_index("s") * 2 + lax.axis_index("c")
    lanes = lax.iota(jnp.int32, 16)
    tilebase = wid * RANGE

    tb = pl.multiple_of(wid * 32, 8)
    pltpu.sync_copy(starts_hbm.at[pl.ds(tb, 32)], sbuf)
    pltpu.sync_copy(ends_hbm.at[pl.ds(tb, 32)], ebuf)
    pltpu.sync_copy(acc_hbm, accbuf)
    idxbuf[pl.ds(SLEN, 16)] = jnp.full((16,), PADV, jnp.int32)

    def set_path():
        bufs = (winbuf0, winbuf1)
        sin = (si0, si1)
        sout = (so0, so1)
        h_out = [None, None]

        def gb(w):
            return pl.multiple_of(tilebase + w * W, 8)

        h_in = [None, None]
        h_in[0] = pltpu.async_copy(x_hbm.at[pl.ds(gb(0), W)], bufs[0],
                                   sin[0])
        for w in range(NWIN):
            cur = w % 2
            nxt = (w + 1) % 2
            h_in[cur].wait()
            if w + 1 < NWIN:
                if h_out[nxt] is not None:
                    h_out[nxt].wait()
                h_in[nxt] = pltpu.async_copy(
                    x_hbm.at[pl.ds(gb(w + 1), W)], bufs[nxt], sin[nxt])
            p_lo = sbuf[pl.ds(w, 16)][0]
            p_hi = ebuf[pl.ds(w, 16)][0]
            _apply_pairs_set(bufs[cur], idxbuf, valbuf, sidx_hbm, sval_hbm,
                             gb(w), p_lo, p_hi)
            h_out[cur] = pltpu.async_copy(
                bufs[cur], out_hbm.at[pl.ds(gb(w), W)], sout[cur])
        h_out[0].wait()
        h_out[1].wait()

    def add_window_body(w, carry_w):
        gbase = pl.multiple_of(tilebase + w * W, 8)
        pltpu.sync_copy(x_hbm.at[pl.ds(gbase, W)], winbuf0)

        p_lo = sbuf[pl.ds(w, 16)][0]
        p_hi = ebuf[pl.ds(w, 16)][0]
        a_start = pl.multiple_of(p_lo & ~7, 8)
        nb = (p_hi - a_start + CAP - 1) // CAP

        def batch_body(b, carry):
            src0 = pl.multiple_of(a_start + b * CAP, 8)
            src = pl.multiple_of(jnp.minimum(src0, SRCMAX), 8)
            delta = src0 - src
            pltpu.sync_copy(sidx_hbm.at[pl.ds(src, SLEN)],
                            idxbuf.at[pl.ds(0, SLEN)])
            pltpu.sync_copy(sval_hbm.at[pl.ds(src, SLEN)],
                            valbuf.at[pl.ds(0, SLEN)])
            todo = jnp.minimum(p_hi - src0, CAP)
            nch = (todo + 15) // 16

            def chunk_body(c, carry2):
                i0 = delta + c * 16
                a = idxbuf[pl.ds(i0, 16)]
                an = idxbuf[pl.ds(i0 + 1, 16)]
                v = valbuf[pl.ds(i0, 16)]
                local = a - gbase
                inwin = (local >= 0) & (local < W)
                runend = a != an
                lclamp = jnp.minimum(jnp.maximum(local, 0), W - 1)
                sv_ = v
                for d in (1, 2, 4, 8):
                    srcl = jnp.maximum(lanes - d, 0)
                    vsh = _gather16(sv_, srcl)
                    ash = _gather16(a, srcl)
                    cond = (lanes >= d) & (a == ash)
                    sv_ = jnp.where(cond, sv_ + vsh, sv_)
                carry_val, carry_idx = carry2
                first_idx = jnp.sum(jnp.where(lanes == 0, a, 0), axis=0)
                cont = carry_idx == first_idx
                headmask = (a == first_idx) & cont
                sv_ = jnp.where(headmask, sv_ + carry_val, sv_)
                plsc.addupdate_scatter(winbuf0, [lclamp], sv_,
                                       mask=runend & inwin)
                last_val = jnp.sum(jnp.where(lanes == 15, sv_,
                                             jnp.float32(0.0)), axis=0)
                last_idx = jnp.sum(jnp.where(lanes == 15, a, 0), axis=0)
                last_end = jnp.sum(jnp.where(lanes == 15,
                                             runend.astype(jnp.int32),
                                             0), axis=0)
                new_cv = jnp.where(last_end == 1, jnp.float32(0.0),
                                   last_val)
                new_ci = jnp.where(last_end == 1, jnp.int32(-1),
                                   last_idx)
                return (new_cv, new_ci)

            return lax.fori_loop(0, nch, chunk_body, carry)

        carry0 = (jnp.float32(0.0), jnp.int32(-1))
        lax.fori_loop(0, nb, batch_body, carry0)
        pltpu.sync_copy(winbuf0, out_hbm.at[pl.ds(gbase, W)])
        return carry_w

    acc = accbuf[pl.ds(0, 16)][0]
    lax.cond(acc != 0,
             lambda: (lax.fori_loop(0, NWIN, add_window_body, 0), None)[1],
             set_path)


def _make_sc_kernel():
    mesh = plsc.VectorSubcoreMesh(core_axis_name="c", subcore_axis_name="s")
    return pl.kernel(
        _scatter_body,
        out_type=jax.ShapeDtypeStruct((N,), jnp.float32),
        mesh=mesh,
        compiler_params=pltpu.CompilerParams(needs_layout_passes=False),
        scratch_types=[
            pltpu.VMEM((W,), jnp.float32),
            pltpu.VMEM((W,), jnp.float32),
            pltpu.VMEM((SLEN + 16,), jnp.int32),
            pltpu.VMEM((SLEN + 16,), jnp.float32),
            pltpu.VMEM((32,), jnp.int32),
            pltpu.VMEM((32,), jnp.int32),
            pltpu.VMEM((16,), jnp.int32),
            pltpu.SemaphoreType.DMA,
            pltpu.SemaphoreType.DMA,
            pltpu.SemaphoreType.DMA,
            pltpu.SemaphoreType.DMA,
        ],
    )


def kernel(x, index, values, accumulate):
    idx32 = index.astype(jnp.int32)
    s_idx, s_val = lax.sort((idx32, values), num_keys=1, is_stable=False)

    win_starts = (jnp.arange(TOTWIN, dtype=jnp.int32) * W)
    b = jnp.searchsorted(s_idx, win_starts, side="left").astype(jnp.int32)
    e = jnp.concatenate([b[1:], jnp.array([M], jnp.int32)])
    starts = jnp.pad(b.reshape(NW, NWIN), ((0, 0), (0, 16))).reshape(-1)
    ends = jnp.pad(e.reshape(NW, NWIN), ((0, 0), (0, 16))).reshape(-1)
    accv = jnp.full((16,), jnp.asarray(accumulate).astype(jnp.int32))

    sc_k = _make_sc_kernel()
    return sc_k(x, s_idx, s_val, starts, ends, accv)


# speedup vs baseline: 1.0432x; 1.0149x over previous
import jax
import jax.numpy as jnp
from jax import lax
from jax.experimental import pallas as pl
from jax.experimental.pallas import tpu as pltpu
from jax.experimental.pallas import tpu_sc as plsc

N = 16777216
M = 1048576
NW = 32
RANGE = N // NW
W = 32768
NWIN = RANGE // W
TOTWIN = NW * NWIN
CAP = 4096
SLEN = CAP + 16
SRCMAX = M - SLEN
PADV = 0x7F000000


def _gather16(vec, idxs):
    dnums = lax.GatherDimensionNumbers(
        offset_dims=(), collapsed_slice_dims=(0,), start_index_map=(0,))
    return lax.gather(vec, idxs[:, None], dimension_numbers=dnums,
                      slice_sizes=(1,),
                      mode=lax.GatherScatterMode.PROMISE_IN_BOUNDS)


def _apply_pairs_set(winbuf, idxbuf, valbuf, sidx_hbm, sval_hbm,
                     gbase, p_lo, p_hi):
    a_start = pl.multiple_of(p_lo & ~7, 8)
    nb = (p_hi - a_start + CAP - 1) // CAP

    def batch_body(b, carry):
        src0 = pl.multiple_of(a_start + b * CAP, 8)
        src = pl.multiple_of(jnp.minimum(src0, SRCMAX), 8)
        delta = src0 - src
        pltpu.sync_copy(sidx_hbm.at[pl.ds(src, SLEN)],
                        idxbuf.at[pl.ds(0, SLEN)])
        pltpu.sync_copy(sval_hbm.at[pl.ds(src, SLEN)],
                        valbuf.at[pl.ds(0, SLEN)])
        todo = jnp.minimum(p_hi - src0, CAP)
        nch = (todo + 15) // 16

        def chunk_body(c, carry2):
            i0 = delta + c * 16
            a = idxbuf[pl.ds(i0, 16)]
            an = idxbuf[pl.ds(i0 + 1, 16)]
            v = valbuf[pl.ds(i0, 16)]
            local = a - gbase
            inwin = (local >= 0) & (local < W)
            runend = a != an
            lclamp = jnp.minimum(jnp.maximum(local, 0), W - 1)
            plsc.store_scatter(winbuf, [lclamp], v, mask=runend & inwin)
            return carry2

        return lax.fori_loop(0, nch, chunk_body, carry)

    lax.fori_loop(0, nb, batch_body, 0)


def _scatter_body(x_hbm, sidx_hbm, sval_hbm, starts_hbm, ends_hbm, acc_hbm,
                  out_hbm, winbuf0, winbuf1, idxbuf, valbuf, sbuf, ebuf,
                  accbuf, si0, si1, so0, so1):
    wid = lax.axis_index("s") * 2 + lax.axis_index("c")
    lanes = lax.iota(jnp.int32, 16)
    tilebase = wid * RANGE

    tb = pl.multiple_of(wid * 32, 8)
    pltpu.sync_copy(starts_hbm.at[pl.ds(tb, 32)], sbuf)
    pltpu.sync_copy(ends_hbm.at[pl.ds(tb, 32)], ebuf)
    pltpu.sync_copy(acc_hbm, accbuf)
    idxbuf[pl.ds(SLEN, 16)] = jnp.full((16,), PADV, jnp.int32)

    def set_path():
        bufs = (winbuf0, winbuf1)
        sin = (si0, si1)
        sout = (so0, so1)
        h_out = [None, None]

        def gb(w):
            return pl.multiple_of(tilebase + w * W, 8)

        h_in = [None, None]
        h_in[0] = pltpu.async_copy(x_hbm.at[pl.ds(gb(0), W)], bufs[0],
                                   sin[0])
        for w in range(NWIN):
            cur = w % 2
            nxt = (w + 1) % 2
            h_in[cur].wait()
            if w + 1 < NWIN:
                if h_out[nxt] is not None:
                    h_out[nxt].wait()
                h_in[nxt] = pltpu.async_copy(
                    x_hbm.at[pl.ds(gb(w + 1), W)], bufs[nxt], sin[nxt])
            p_lo = sbuf[pl.ds(w, 16)][0]
            p_hi = ebuf[pl.ds(w, 16)][0]
            _apply_pairs_set(bufs[cur], idxbuf, valbuf, sidx_hbm, sval_hbm,
                             gb(w), p_lo, p_hi)
            h_out[cur] = pltpu.async_copy(
                bufs[cur], out_hbm.at[pl.ds(gb(w), W)], sout[cur])
        h_out[0].wait()
        h_out[1].wait()

    def add_window_body(w, carry_w):
        gbase = pl.multiple_of(tilebase + w * W, 8)
        pltpu.sync_copy(x_hbm.at[pl.ds(gbase, W)], winbuf0)

        p_lo = sbuf[pl.ds(w, 16)][0]
        p_hi = ebuf[pl.ds(w, 16)][0]
        a_start = pl.multiple_of(p_lo & ~7, 8)
        nb = (p_hi - a_start + CAP - 1) // CAP

        def batch_body(b, carry):
            src0 = pl.multiple_of(a_start + b * CAP, 8)
            src = pl.multiple_of(jnp.minimum(src0, SRCMAX), 8)
            delta = src0 - src
            pltpu.sync_copy(sidx_hbm.at[pl.ds(src, SLEN)],
                            idxbuf.at[pl.ds(0, SLEN)])
            pltpu.sync_copy(sval_hbm.at[pl.ds(src, SLEN)],
                            valbuf.at[pl.ds(0, SLEN)])
            todo = jnp.minimum(p_hi - src0, CAP)
            nch = (todo + 15) // 16

            def chunk_body(c, carry2):
                i0 = delta + c * 16
                a = idxbuf[pl.ds(i0, 16)]
                an = idxbuf[pl.ds(i0 + 1, 16)]
                v = valbuf[pl.ds(i0, 16)]
                local = a - gbase
                inwin = (local >= 0) & (local < W)
                runend = a != an
                lclamp = jnp.minimum(jnp.maximum(local, 0), W - 1)
                sv_ = v
                for d in (1, 2, 4, 8):
                    srcl = jnp.maximum(lanes - d, 0)
                    vsh = _gather16(sv_, srcl)
                    ash = _gather16(a, srcl)
                    cond = (lanes >= d) & (a == ash)
                    sv_ = jnp.where(cond, sv_ + vsh, sv_)
                carry_val, carry_idx = carry2
                first_idx = jnp.sum(jnp.where(lanes == 0, a, 0), axis=0)
                cont = carry_idx == first_idx
                headmask = (a == first_idx) & cont
                sv_ = jnp.where(headmask, sv_ + carry_val, sv_)
                plsc.addupdate_scatter(winbuf0, [lclamp], sv_,
                                       mask=runend & inwin)
                last_val = jnp.sum(jnp.where(lanes == 15, sv_,
                                             jnp.float32(0.0)), axis=0)
                last_idx = jnp.sum(jnp.where(lanes == 15, a, 0), axis=0)
                last_end = jnp.sum(jnp.where(lanes == 15,
                                             runend.astype(jnp.int32),
                                             0), axis=0)
                new_cv = jnp.where(last_end == 1, jnp.float32(0.0),
                                   last_val)
                new_ci = jnp.where(last_end == 1, jnp.int32(-1),
                                   last_idx)
                return (new_cv, new_ci)

            return lax.fori_loop(0, nch, chunk_body, carry)

        carry0 = (jnp.float32(0.0), jnp.int32(-1))
        lax.fori_loop(0, nb, batch_body, carry0)
        pltpu.sync_copy(winbuf0, out_hbm.at[pl.ds(gbase, W)])
        return carry_w

    acc = accbuf[pl.ds(0, 16)][0]
    lax.cond(acc != 0,
             lambda: (lax.fori_loop(0, NWIN, add_window_body, 0), None)[1],
             set_path)


def _make_sc_kernel():
    mesh = plsc.VectorSubcoreMesh(core_axis_name="c", subcore_axis_name="s")
    return pl.kernel(
        _scatter_body,
        out_type=jax.ShapeDtypeStruct((N,), jnp.float32),
        mesh=mesh,
        compiler_params=pltpu.CompilerParams(needs_layout_passes=False),
        scratch_types=[
            pltpu.VMEM((W,), jnp.float32),
            pltpu.VMEM((W,), jnp.float32),
            pltpu.VMEM((SLEN + 16,), jnp.int32),
            pltpu.VMEM((SLEN + 16,), jnp.float32),
            pltpu.VMEM((32,), jnp.int32),
            pltpu.VMEM((32,), jnp.int32),
            pltpu.VMEM((16,), jnp.int32),
            pltpu.SemaphoreType.DMA,
            pltpu.SemaphoreType.DMA,
            pltpu.SemaphoreType.DMA,
            pltpu.SemaphoreType.DMA,
        ],
    )


def kernel(x, index, values, accumulate):
    idx32 = index.astype(jnp.int32)
    s_idx, s_val = lax.sort((idx32, values), num_keys=1, is_stable=False)

    win_starts = (jnp.arange(TOTWIN, dtype=jnp.int32) * W)
    b = jnp.searchsorted(s_idx, win_starts, side="left").astype(jnp.int32)
    e = jnp.concatenate([b[1:], jnp.array([M], jnp.int32)])
    starts = jnp.pad(b.reshape(NW, NWIN), ((0, 0), (0, 16))).reshape(-1)
    ends = jnp.pad(e.reshape(NW, NWIN), ((0, 0), (0, 16))).reshape(-1)
    accv = jnp.full((16,), jnp.asarray(accumulate).astype(jnp.int32))

    sc_k = _make_sc_kernel()
    return sc_k(x, s_idx, s_val, starts, ends, accv)
